# baseline (speedup 1.0000x reference)
import functools

import jax
import jax.numpy as jnp
from jax.experimental import pallas as pl
from jax.experimental.pallas import tpu as pltpu

_MARGIN = 0.4
_EPS = 1e-12


def _dist_tile(fT_ref, xj_ref):
    xiT = fT_ref[...]
    xj = xj_ref[...]
    sqi = jnp.sum(xiT * xiT, axis=0, keepdims=True)
    sqj = jnp.sum(xj * xj, axis=1, keepdims=True)
    prod = jax.lax.dot_general(
        xj, xiT, (((1,), (0,)), ((), ())),
        preferred_element_type=jnp.float32)
    d2 = sqj + sqi - 2.0 * prod
    return jnp.sqrt(jnp.maximum(d2, 0.0) + _EPS)


def _s_kernel(fT_ref, xj_ref, li_ref, lj_ref, s_ref):
    j = pl.program_id(1)
    d = _dist_tile(fT_ref, xj_ref)
    neg = lj_ref[...] != li_ref[...]
    e = jnp.where(neg, jnp.exp(_MARGIN - d), 0.0)
    part = jnp.sum(e, axis=0, keepdims=True)[None]

    @pl.when(j == 0)
    def _():
        s_ref[...] = part

    @pl.when(j > 0)
    def _():
        s_ref[...] = s_ref[...] + part


def _loss_kernel(fT_ref, xj_ref, li_ref, lj_ref, si_ref, sj_ref,
                 loss_ref, cnt_ref):
    j = pl.program_id(1)
    d = _dist_tile(fT_ref, xj_ref)
    pos = lj_ref[...] == li_ref[...]
    jv = jnp.log(sj_ref[...] + si_ref[...]) + d
    jv = jnp.where(pos, jnp.maximum(jv, 0.0), 0.0)
    lpart = jnp.sum(jv * jv, axis=0, keepdims=True)[None]
    cpart = jnp.sum(jnp.where(pos, 1.0, 0.0), axis=0, keepdims=True)[None]

    @pl.when(j == 0)
    def _():
        loss_ref[...] = lpart
        cnt_ref[...] = cpart

    @pl.when(j > 0)
    def _():
        loss_ref[...] = loss_ref[...] + lpart
        cnt_ref[...] = cnt_ref[...] + cpart


@jax.jit
def kernel(features, labels):
    N, D = features.shape
    BM, BN = 1024, 512
    n_i, n_j = N // BM, N // BN

    labels = labels.astype(jnp.int32)
    fT = features.T
    lab_row = labels.reshape(1, N)
    lab_col = labels.reshape(N, 1)

    grid = (n_i, n_j)
    params = pltpu.CompilerParams(
        dimension_semantics=("parallel", "arbitrary"))

    fT_spec = pl.BlockSpec((D, BM), lambda i, j: (0, i))
    xj_spec = pl.BlockSpec((BN, D), lambda i, j: (j, 0))
    li_spec = pl.BlockSpec((1, BM), lambda i, j: (0, i))
    lj_spec = pl.BlockSpec((BN, 1), lambda i, j: (j, 0))
    acc_spec = pl.BlockSpec((1, 1, BM), lambda i, j: (i, 0, 0))
    acc_shape = jax.ShapeDtypeStruct((n_i, 1, BM), jnp.float32)

    s = pl.pallas_call(
        _s_kernel,
        grid=grid,
        in_specs=[fT_spec, xj_spec, li_spec, lj_spec],
        out_specs=acc_spec,
        out_shape=acc_shape,
        compiler_params=params,
    )(fT, features, lab_row, lab_col)

    s_row = s.reshape(1, N)
    s_col = s.reshape(N, 1)

    loss_rows, cnt_rows = pl.pallas_call(
        _loss_kernel,
        grid=grid,
        in_specs=[fT_spec, xj_spec, li_spec, lj_spec,
                  pl.BlockSpec((1, BM), lambda i, j: (0, i)),
                  pl.BlockSpec((BN, 1), lambda i, j: (j, 0))],
        out_specs=[acc_spec, acc_spec],
        out_shape=[acc_shape, acc_shape],
        compiler_params=params,
    )(fT, features, lab_row, lab_col, s_row, s_col)

    return jnp.sum(loss_rows) / (2.0 * jnp.sum(cnt_rows))


# speedup vs baseline: 1.2655x; 1.2655x over previous
import functools

import jax
import jax.numpy as jnp
from jax.experimental import pallas as pl
from jax.experimental.pallas import tpu as pltpu

_MARGIN = 0.4
_EPS = 1e-12


def _sq_kernel(fT_ref, sq_ref):
    xT = fT_ref[...]
    sq_ref[...] = jnp.sum(xT * xT, axis=0, keepdims=True)[None]


def _dist_tile(fT_ref, xj_ref, sqi_ref, sqj_ref):
    prod = jax.lax.dot_general(
        xj_ref[...], fT_ref[...], (((1,), (0,)), ((), ())),
        preferred_element_type=jnp.float32)
    d2 = (sqj_ref[...] + sqi_ref[...]) - 2.0 * prod
    d2 = jnp.maximum(d2, 0.0) + _EPS
    return d2 * jax.lax.rsqrt(d2)


def _s_kernel(fT_ref, xj_ref, li_ref, lj_ref, sqi_ref, sqj_ref, s_ref):
    j = pl.program_id(1)
    d = _dist_tile(fT_ref, xj_ref, sqi_ref, sqj_ref)
    neg = lj_ref[...] != li_ref[...]
    e = jnp.where(neg, jnp.exp(_MARGIN - d), 0.0)
    part = jnp.sum(e, axis=0, keepdims=True)[None]

    @pl.when(j == 0)
    def _():
        s_ref[...] = part

    @pl.when(j > 0)
    def _():
        s_ref[...] = s_ref[...] + part


def _loss_kernel(fT_ref, xj_ref, li_ref, lj_ref, sqi_ref, sqj_ref,
                 si_ref, sj_ref, loss_ref, cnt_ref):
    j = pl.program_id(1)
    d = _dist_tile(fT_ref, xj_ref, sqi_ref, sqj_ref)
    pos = lj_ref[...] == li_ref[...]
    jv = jnp.log(sj_ref[...] + si_ref[...]) + d
    jv = jnp.where(pos, jnp.maximum(jv, 0.0), 0.0)
    lpart = jnp.sum(jv * jv, axis=0, keepdims=True)[None]
    cpart = jnp.sum(jnp.where(pos, 1.0, 0.0), axis=0, keepdims=True)[None]

    @pl.when(j == 0)
    def _():
        loss_ref[...] = lpart
        cnt_ref[...] = cpart

    @pl.when(j > 0)
    def _():
        loss_ref[...] = loss_ref[...] + lpart
        cnt_ref[...] = cnt_ref[...] + cpart


@jax.jit
def kernel(features, labels):
    N, D = features.shape
    BM, BN = 1024, 512
    n_i, n_j = N // BM, N // BN

    labels = labels.astype(jnp.int32)
    fT = features.T
    lab_row = labels.reshape(1, N)
    lab_col = labels.reshape(N, 1)

    BT = 2048
    sq = pl.pallas_call(
        _sq_kernel,
        grid=(N // BT,),
        in_specs=[pl.BlockSpec((D, BT), lambda i: (0, i))],
        out_specs=pl.BlockSpec((1, 1, BT), lambda i: (i, 0, 0)),
        out_shape=jax.ShapeDtypeStruct((N // BT, 1, BT), jnp.float32),
        compiler_params=pltpu.CompilerParams(
            dimension_semantics=("parallel",)),
    )(fT)
    sq_row = sq.reshape(1, N)
    sq_col = sq.reshape(N, 1)

    grid = (n_i, n_j)
    params = pltpu.CompilerParams(
        dimension_semantics=("parallel", "arbitrary"))

    fT_spec = pl.BlockSpec((D, BM), lambda i, j: (0, i))
    xj_spec = pl.BlockSpec((BN, D), lambda i, j: (j, 0))
    li_spec = pl.BlockSpec((1, BM), lambda i, j: (0, i))
    lj_spec = pl.BlockSpec((BN, 1), lambda i, j: (j, 0))
    sqi_spec = pl.BlockSpec((1, BM), lambda i, j: (0, i))
    sqj_spec = pl.BlockSpec((BN, 1), lambda i, j: (j, 0))
    acc_spec = pl.BlockSpec((1, 1, BM), lambda i, j: (i, 0, 0))
    acc_shape = jax.ShapeDtypeStruct((n_i, 1, BM), jnp.float32)

    s = pl.pallas_call(
        _s_kernel,
        grid=grid,
        in_specs=[fT_spec, xj_spec, li_spec, lj_spec, sqi_spec, sqj_spec],
        out_specs=acc_spec,
        out_shape=acc_shape,
        compiler_params=params,
    )(fT, features, lab_row, lab_col, sq_row, sq_col)

    s_row = s.reshape(1, N)
    s_col = s.reshape(N, 1)

    loss_rows, cnt_rows = pl.pallas_call(
        _loss_kernel,
        grid=grid,
        in_specs=[fT_spec, xj_spec, li_spec, lj_spec, sqi_spec, sqj_spec,
                  pl.BlockSpec((1, BM), lambda i, j: (0, i)),
                  pl.BlockSpec((BN, 1), lambda i, j: (j, 0))],
        out_specs=[acc_spec, acc_spec],
        out_shape=[acc_shape, acc_shape],
        compiler_params=params,
    )(fT, features, lab_row, lab_col, sq_row, sq_col, s_row, s_col)

    return jnp.sum(loss_rows) / (2.0 * jnp.sum(cnt_rows))


# speedup vs baseline: 1.3269x; 1.0485x over previous
import functools

import jax
import jax.numpy as jnp
from jax.experimental import pallas as pl
from jax.experimental.pallas import tpu as pltpu

_MARGIN = 0.4
_EPS = 1e-12
_LOG2E = 1.4426950408889634
_LN2 = 0.6931471805599453
_C = 128


def _sq_kernel(x_ref, lab_ref, sq_ref, cnt_ref):
    x = x_ref[...]
    ones = jnp.ones((1, x.shape[1]), jnp.float32)
    sq_ref[...] = jax.lax.dot_general(
        ones, x * x, (((1,), (1,)), ((), ())),
        preferred_element_type=jnp.float32,
        precision=jax.lax.Precision.HIGHEST)[None]
    cls = jax.lax.broadcasted_iota(jnp.int32, (1, _C), 1)
    hit = jnp.where(lab_ref[...] == cls, 1.0, 0.0)
    cnt_ref[...] = jnp.sum(hit, axis=0, keepdims=True)[None]


def _dist_tile(fT_ref, xj_ref, sqi_ref, sqj_ref):
    prod = jax.lax.dot_general(
        xj_ref[...], fT_ref[...], (((1,), (0,)), ((), ())),
        preferred_element_type=jnp.float32)
    d2 = (sqj_ref[...] + sqi_ref[...]) + prod
    d2 = jnp.maximum(d2, 0.0) + _EPS
    return d2 * jax.lax.rsqrt(d2)


def _s_kernel(fT_ref, xj_ref, li_ref, lj_ref, sqi_ref, sqj_ref, s_ref):
    j = pl.program_id(1)
    d = _dist_tile(fT_ref, xj_ref, sqi_ref, sqj_ref)
    neg = lj_ref[...] != li_ref[...]
    e = jnp.exp2(_MARGIN * _LOG2E - _LOG2E * d)
    e = jnp.where(neg, e, 0.0)
    part = jnp.sum(e, axis=0, keepdims=True)[None]

    @pl.when(j == 0)
    def _():
        s_ref[...] = part

    @pl.when(j > 0)
    def _():
        s_ref[...] = s_ref[...] + part


def _loss_kernel(fT_ref, xj_ref, li_ref, lj_ref, sqi_ref, sqj_ref,
                 si_ref, sj_ref, loss_ref):
    j = pl.program_id(1)
    d = _dist_tile(fT_ref, xj_ref, sqi_ref, sqj_ref)
    pos = lj_ref[...] == li_ref[...]
    jv = jnp.log(sj_ref[...] + si_ref[...]) + d
    jv = jnp.where(pos, jnp.maximum(jv, 0.0), 0.0)
    lpart = jnp.sum(jv * jv, axis=0, keepdims=True)[None]

    @pl.when(j == 0)
    def _():
        loss_ref[...] = lpart

    @pl.when(j > 0)
    def _():
        loss_ref[...] = loss_ref[...] + lpart


@jax.jit
def kernel(features, labels):
    N, D = features.shape
    BM, BN = 1024, 512
    n_i, n_j = N // BM, N // BN

    labels = labels.astype(jnp.int32)
    lab_row = labels.reshape(1, N)
    lab_col = labels.reshape(N, 1)

    f_bf = features.astype(jnp.bfloat16)
    fTn2 = (-2.0 * features).T.astype(jnp.bfloat16)

    BT = 2048
    sq, cnt = pl.pallas_call(
        _sq_kernel,
        grid=(N // BT,),
        in_specs=[pl.BlockSpec((BT, D), lambda i: (i, 0)),
                  pl.BlockSpec((BT, 1), lambda i: (i, 0))],
        out_specs=[pl.BlockSpec((1, 1, BT), lambda i: (i, 0, 0)),
                   pl.BlockSpec((1, 1, _C), lambda i: (i, 0, 0))],
        out_shape=[jax.ShapeDtypeStruct((N // BT, 1, BT), jnp.float32),
                   jax.ShapeDtypeStruct((N // BT, 1, _C), jnp.float32)],
        compiler_params=pltpu.CompilerParams(
            dimension_semantics=("parallel",)),
    )(features, lab_col)
    sq_row = sq.reshape(1, N)
    sq_col = sq.reshape(N, 1)
    counts = jnp.sum(cnt, axis=0)
    num_pos = jnp.sum(counts * counts)

    grid = (n_i, n_j)
    params = pltpu.CompilerParams(
        dimension_semantics=("parallel", "arbitrary"))

    fT_spec = pl.BlockSpec((D, BM), lambda i, j: (0, i))
    xj_spec = pl.BlockSpec((BN, D), lambda i, j: (j, 0))
    li_spec = pl.BlockSpec((1, BM), lambda i, j: (0, i))
    lj_spec = pl.BlockSpec((BN, 1), lambda i, j: (j, 0))
    sqi_spec = pl.BlockSpec((1, BM), lambda i, j: (0, i))
    sqj_spec = pl.BlockSpec((BN, 1), lambda i, j: (j, 0))
    acc_spec = pl.BlockSpec((1, 1, BM), lambda i, j: (i, 0, 0))
    acc_shape = jax.ShapeDtypeStruct((n_i, 1, BM), jnp.float32)

    s = pl.pallas_call(
        _s_kernel,
        grid=grid,
        in_specs=[fT_spec, xj_spec, li_spec, lj_spec, sqi_spec, sqj_spec],
        out_specs=acc_spec,
        out_shape=acc_shape,
        compiler_params=params,
    )(fTn2, f_bf, lab_row, lab_col, sq_row, sq_col)

    s_row = s.reshape(1, N)
    s_col = s.reshape(N, 1)

    loss_rows = pl.pallas_call(
        _loss_kernel,
        grid=grid,
        in_specs=[fT_spec, xj_spec, li_spec, lj_spec, sqi_spec, sqj_spec,
                  pl.BlockSpec((1, BM), lambda i, j: (0, i)),
                  pl.BlockSpec((BN, 1), lambda i, j: (j, 0))],
        out_specs=acc_spec,
        out_shape=acc_shape,
        compiler_params=params,
    )(fTn2, f_bf, lab_row, lab_col, sq_row, sq_col, s_row, s_col)

    return jnp.sum(loss_rows) / (2.0 * num_pos)
